# Initial kernel scaffold; baseline (speedup 1.0000x reference)
import jax
import jax.numpy as jnp
from jax.experimental import pallas as pl


def kernel(x, y, z):
    raise NotImplementedError("write your pallas kernel here")


# speedup vs baseline: 7.4702x; 7.4702x over previous
import jax
import jax.numpy as jnp
from jax.experimental import pallas as pl


def _body(xa_ref, xb_ref, z_ref, y_ref, xo_ref, yo_ref, zo_ref):
    xo_ref[0] = xa_ref[0, 0]
    xo_ref[1] = xb_ref[0, 0]
    zo_ref[0] = z_ref[0, 0]
    yo_ref[:, :, 0] = y_ref[:, :, 1]
    yo_ref[:, :, 1] = y_ref[:, :, 0]


def kernel(x, y, z):
    R = 256
    G = 2048 // R
    YR = 4096 // G

    grid = (G,)
    out_shapes = (
        jax.ShapeDtypeStruct((2, 2048, 1024), jnp.float32),
        jax.ShapeDtypeStruct((4, 4096, 2), jnp.float32),
        jax.ShapeDtypeStruct((1, 2048, 1024), jnp.float32),
    )
    in_specs = [
        pl.BlockSpec((1, 1, R, 1024), lambda g: (2, 0, g, 0)),
        pl.BlockSpec((1, 1, R, 1024), lambda g: (1, 1, g, 0)),
        pl.BlockSpec((1, 1, R, 1024), lambda g: (0, 2, g, 0)),
        pl.BlockSpec((4, YR, 128), lambda g: (0, g, 0)),
    ]
    out_specs = (
        pl.BlockSpec((2, R, 1024), lambda g: (0, g, 0)),
        pl.BlockSpec((4, YR, 2), lambda g: (0, g, 0)),
        pl.BlockSpec((1, R, 1024), lambda g: (0, g, 0)),
    )
    return pl.pallas_call(
        _body,
        grid=grid,
        in_specs=in_specs,
        out_specs=out_specs,
        out_shape=out_shapes,
    )(x, x, z, y)
